# speedup vs baseline: 32.0042x; 32.0042x over previous
import jax
import jax.numpy as jnp
from jax import lax
from jax.experimental import pallas as pl
from jax.experimental.pallas import tpu as pltpu

NEG_SLOPE = 0.01
NB = 128


def _lrelu(v):
    return jnp.where(v > 0, v, NEG_SLOPE * v)


def _fused_kernel(xr_ref, w1_ref, b1_ref, w2_ref, b2_ref,
                  w1pr_ref, bf1_ref, w2p_ref, bf2_ref,
                  o_ref, xrot_ref, p1rot_ref):
    for ci in range(3):
        for mi in range(4):
            for cr in range(2):
                xrot_ref[ci, mi, cr] = xr_ref[ci, :, mi, cr:cr + 7, :]

    def c1_body(co, carry):
        accs = [jnp.zeros((28, 7, NB), jnp.float32) for _ in range(4)]
        for ci in range(3):
            for j in range(5):
                for i in range(5):
                    for mo in range(4):
                        mi = (mo + j) % 4
                        cr = (mo + j) // 4
                        w = w1_ref[co * 75 + ci * 25 + i * 5 + j]
                        accs[mo] = accs[mo] + w * xrot_ref[ci, mi, cr,
                                                           i:i + 28, :, :]
        b = b1_ref[co]
        for u in range(2):
            pw = jnp.maximum(accs[2 * u], accs[2 * u + 1])
            ph = jnp.max(pw.reshape(14, 2, 7, NB), axis=1)
            a = _lrelu(ph + b)
            for cr in range(3):
                p1rot_ref[co, u, cr] = a[:, cr:cr + 5, :]
        return carry

    lax.fori_loop(0, 6, c1_body, 0)

    def c2_body(co, h_acc):
        accs = [jnp.zeros((10, 5, NB), jnp.float32) for _ in range(2)]
        for ci in range(6):
            for j in range(5):
                for i in range(5):
                    for po in range(2):
                        ui = (po + j) % 2
                        cr = (po + j) // 2
                        w = w2_ref[co * 150 + ci * 25 + i * 5 + j]
                        accs[po] = accs[po] + w * p1rot_ref[ci, ui, cr,
                                                            i:i + 10, :, :]
        pw = jnp.maximum(accs[0], accs[1])
        ph = jnp.max(pw.reshape(5, 2, 5, NB), axis=1)
        p2 = _lrelu(ph + b2_ref[co])
        for r in range(5):
            h_acc = h_acc + lax.dot_general(
                w1pr_ref[co, r], p2[r],
                (((1,), (0,)), ((), ())),
                preferred_element_type=jnp.float32)
        return h_acc

    h = lax.fori_loop(0, 16, c2_body, jnp.zeros((8, NB), jnp.float32))
    h = _lrelu(h + bf1_ref[...])

    z = lax.dot_general(w2p_ref[...], h, (((1,), (0,)), ((), ())),
                        preferred_element_type=jnp.float32) + bf2_ref[...]
    o_ref[...] = (1.0 / (1.0 + jnp.exp(-z))).astype(o_ref.dtype)


def _forward_impl(packed, xr, out_pad):
    n_pad = xr.shape[-1]
    smem = pl.BlockSpec(memory_space=pltpu.MemorySpace.SMEM)
    grid_spec = pltpu.PrefetchScalarGridSpec(
        num_scalar_prefetch=0,
        grid=(n_pad // NB,),
        in_specs=[
            pl.BlockSpec((3, 32, 4, 8, NB), lambda b: (0, 0, 0, 0, b)),
            smem, smem, smem, smem,
            pl.BlockSpec((16, 5, 8, 5), lambda b: (0, 0, 0, 0)),
            pl.BlockSpec((8, NB), lambda b: (0, 0)),
            pl.BlockSpec((out_pad, 8), lambda b: (0, 0)),
            pl.BlockSpec((out_pad, NB), lambda b: (0, 0)),
        ],
        out_specs=pl.BlockSpec((out_pad, NB), lambda b: (0, b)),
        scratch_shapes=[
            pltpu.VMEM((3, 4, 2, 32, 7, NB), jnp.float32),
            pltpu.VMEM((6, 2, 3, 14, 5, NB), jnp.float32),
        ],
    )
    return pl.pallas_call(
        _fused_kernel,
        out_shape=jax.ShapeDtypeStruct((out_pad, n_pad), jnp.float32),
        grid_spec=grid_spec,
        compiler_params=pltpu.CompilerParams(
            dimension_semantics=("parallel",),
            vmem_limit_bytes=64 * 1024 * 1024,
        ),
    )(xr, packed["w1"], packed["b1"], packed["w2"], packed["b2"],
      packed["w1pr"], packed["bf1"], packed["w2p"], packed["bf2"])


_forward = jax.jit(_forward_impl, static_argnames=("out_pad",))


def _pack(w_conv1, b_conv1, w_conv2, b_conv2, w_fc1, b_fc1, w_fc2, b_fc2,
          out_pad):
    f32 = jnp.float32
    out_size = w_fc2.shape[0]
    w1pr = jnp.asarray(w_fc1, f32).reshape(5, 16, 5, 5).transpose(1, 2, 0, 3)
    w1pr = jnp.pad(w1pr, ((0, 0), (0, 0), (0, 3), (0, 0)))
    bf1 = jnp.zeros((8, NB), f32).at[:5].set(
        jnp.broadcast_to(jnp.asarray(b_fc1, f32)[:, None], (5, NB)))
    w2p = jnp.zeros((out_pad, 8), f32).at[:out_size, :5].set(
        jnp.asarray(w_fc2, f32))
    bf2 = jnp.zeros((out_pad, NB), f32).at[:out_size].set(
        jnp.broadcast_to(jnp.asarray(b_fc2, f32)[:, None], (out_size, NB)))
    return {
        "w1": jnp.asarray(w_conv1, f32).reshape(-1),
        "b1": jnp.asarray(b_conv1, f32),
        "w2": jnp.asarray(w_conv2, f32).reshape(-1),
        "b2": jnp.asarray(b_conv2, f32),
        "w1pr": w1pr, "bf1": bf1, "w2p": w2p, "bf2": bf2,
    }


def kernel(w_conv1, b_conv1, w_conv2, b_conv2, w_fc1, b_fc1, w_fc2, b_fc2, x):
    n = x.shape[0]
    out_size = w_fc2.shape[0]
    out_pad = max(8, ((out_size + 7) // 8) * 8)
    n_pad = ((n + NB - 1) // NB) * NB

    x4 = jnp.asarray(x, jnp.float32).reshape(n, 3, 32, 32)
    if n_pad != n:
        x4 = jnp.pad(x4, ((0, n_pad - n), (0, 0), (0, 0), (0, 0)))
    xr = x4.reshape(n_pad, 3, 32, 8, 4).transpose(1, 2, 4, 3, 0)

    packed = _pack(w_conv1, b_conv1, w_conv2, b_conv2,
                   w_fc1, b_fc1, w_fc2, b_fc2, out_pad)
    out = _forward(packed, xr, out_pad)
    return out.T[:n, :out_size]


# speedup vs baseline: 141.7255x; 4.4283x over previous
import jax
import jax.numpy as jnp
from jax import lax
from jax.experimental import pallas as pl
from jax.experimental.pallas import tpu as pltpu

NEG_SLOPE = 0.01
NB = 256


def _lrelu(v):
    return jnp.where(v > 0, v, NEG_SLOPE * v)


def _dot(a, b):
    return lax.dot_general(a, b, (((1,), (0,)), ((), ())),
                           preferred_element_type=jnp.float32)


def _fused_kernel(x_ref, w1_ref, a2_ref, a1_ref, b1r_ref, b2r_ref,
                  bf1_ref, w2p_ref, bf2_ref, o_ref, p1_ref):
    def c1_body(r, carry):
        ya = _dot(w1_ref[0], x_ref[2 * r])
        yb = _dot(w1_ref[0], x_ref[2 * r + 1])
        for i in range(1, 5):
            ya = ya + _dot(w1_ref[i], x_ref[2 * r + i])
            yb = yb + _dot(w1_ref[i], x_ref[2 * r + 1 + i])
        pw = jnp.maximum(jnp.maximum(ya[0:84], ya[88:172]),
                         jnp.maximum(yb[0:84], yb[88:172]))
        p1_ref[r] = _lrelu(pw + b1r_ref[...])
        return carry

    lax.fori_loop(0, 14, c1_body, 0)

    def c2_body(r2, h_acc):
        ya = _dot(a2_ref[0], p1_ref[2 * r2])
        yb = _dot(a2_ref[0], p1_ref[2 * r2 + 1])
        for i in range(1, 5):
            ya = ya + _dot(a2_ref[i], p1_ref[2 * r2 + i])
            yb = yb + _dot(a2_ref[i], p1_ref[2 * r2 + 1 + i])
        pw = jnp.maximum(jnp.maximum(ya[0:80], ya[80:160]),
                         jnp.maximum(yb[0:80], yb[80:160]))
        p2 = _lrelu(pw + b2r_ref[...])
        return h_acc + _dot(a1_ref[r2], p2)

    h = lax.fori_loop(0, 5, c2_body, jnp.zeros((8, NB), jnp.float32))
    h = _lrelu(h + bf1_ref[...])

    z = _dot(w2p_ref[...], h) + bf2_ref[...]
    o_ref[...] = (1.0 / (1.0 + jnp.exp(-z))).astype(o_ref.dtype)


def _forward_impl(packed, xt, out_pad):
    n_pad = xt.shape[-1]
    grid_spec = pltpu.PrefetchScalarGridSpec(
        num_scalar_prefetch=0,
        grid=(n_pad // NB,),
        in_specs=[
            pl.BlockSpec((32, 96, NB), lambda b: (0, 0, b)),
            pl.BlockSpec((5, 176, 96), lambda b: (0, 0, 0)),
            pl.BlockSpec((5, 160, 84), lambda b: (0, 0, 0)),
            pl.BlockSpec((5, 8, 80), lambda b: (0, 0, 0)),
            pl.BlockSpec((84, NB), lambda b: (0, 0)),
            pl.BlockSpec((80, NB), lambda b: (0, 0)),
            pl.BlockSpec((8, NB), lambda b: (0, 0)),
            pl.BlockSpec((out_pad, 8), lambda b: (0, 0)),
            pl.BlockSpec((out_pad, NB), lambda b: (0, 0)),
        ],
        out_specs=pl.BlockSpec((out_pad, NB), lambda b: (0, b)),
        scratch_shapes=[
            pltpu.VMEM((14, 84, NB), jnp.float32),
        ],
    )
    return pl.pallas_call(
        _fused_kernel,
        out_shape=jax.ShapeDtypeStruct((out_pad, n_pad), jnp.float32),
        grid_spec=grid_spec,
        compiler_params=pltpu.CompilerParams(
            dimension_semantics=("parallel",),
            vmem_limit_bytes=64 * 1024 * 1024,
        ),
    )(xt, packed["w1"], packed["a2"], packed["a1"], packed["b1r"],
      packed["b2r"], packed["bf1"], packed["w2p"], packed["bf2"])


_forward = jax.jit(_forward_impl, static_argnames=("out_pad",))


def _band1(w, i):
    wi = w[:, :, i, :]
    p = jnp.arange(2)[:, None, None]
    s = jnp.arange(14)[None, :, None]
    win = jnp.arange(32)[None, None, :]
    jm = win - (2 * s + p)
    mask = (jm >= 0) & (jm < 5)
    jc = jnp.clip(jm, 0, 4)
    g = wi[:, :, jc]
    g = jnp.where(mask[None, None], g, 0.0)
    g = g.transpose(2, 0, 3, 1, 4).reshape(2, 84, 96)
    return jnp.pad(g, ((0, 0), (0, 4), (0, 0))).reshape(176, 96)


def _band2(w, i):
    wi = w[:, :, i, :]
    p = jnp.arange(2)[:, None, None]
    t = jnp.arange(5)[None, :, None]
    sin = jnp.arange(14)[None, None, :]
    jm = sin - (2 * t + p)
    mask = (jm >= 0) & (jm < 5)
    jc = jnp.clip(jm, 0, 4)
    g = wi[:, :, jc]
    g = jnp.where(mask[None, None], g, 0.0)
    return g.transpose(2, 0, 3, 1, 4).reshape(160, 84)


def _pack(w_conv1, b_conv1, w_conv2, b_conv2, w_fc1, b_fc1, w_fc2, b_fc2,
          out_pad):
    f32 = jnp.float32
    out_size = w_fc2.shape[0]
    w1 = jnp.asarray(w_conv1, f32)
    w2 = jnp.asarray(w_conv2, f32)
    w1b = jnp.stack([_band1(w1, i) for i in range(5)])
    a2b = jnp.stack([_band2(w2, i) for i in range(5)])
    wf1 = jnp.asarray(w_fc1, f32).reshape(5, 16, 5, 5)
    a1 = wf1.transpose(2, 0, 1, 3).reshape(5, 5, 80)
    a1 = jnp.pad(a1, ((0, 0), (0, 3), (0, 0)))
    b1r = jnp.broadcast_to(
        jnp.repeat(jnp.asarray(b_conv1, f32), 14)[:, None], (84, NB))
    b2r = jnp.broadcast_to(
        jnp.repeat(jnp.asarray(b_conv2, f32), 5)[:, None], (80, NB))
    bf1 = jnp.zeros((8, NB), f32).at[:5].set(
        jnp.broadcast_to(jnp.asarray(b_fc1, f32)[:, None], (5, NB)))
    w2p = jnp.zeros((out_pad, 8), f32).at[:out_size, :5].set(
        jnp.asarray(w_fc2, f32))
    bf2 = jnp.zeros((out_pad, NB), f32).at[:out_size].set(
        jnp.broadcast_to(jnp.asarray(b_fc2, f32)[:, None], (out_size, NB)))
    return {"w1": w1b, "a2": a2b, "a1": a1, "b1r": b1r, "b2r": b2r,
            "bf1": bf1, "w2p": w2p, "bf2": bf2}


def kernel(w_conv1, b_conv1, w_conv2, b_conv2, w_fc1, b_fc1, w_fc2, b_fc2, x):
    n = x.shape[0]
    out_size = w_fc2.shape[0]
    out_pad = max(8, ((out_size + 7) // 8) * 8)
    n_pad = ((n + NB - 1) // NB) * NB

    x4 = jnp.asarray(x, jnp.float32).reshape(n, 3, 32, 32)
    if n_pad != n:
        x4 = jnp.pad(x4, ((0, n_pad - n), (0, 0), (0, 0), (0, 0)))
    xt = x4.transpose(2, 1, 3, 0).reshape(32, 96, n_pad)

    packed = _pack(w_conv1, b_conv1, w_conv2, b_conv2,
                   w_fc1, b_fc1, w_fc2, b_fc2, out_pad)
    out = _forward(packed, xt, out_pad)
    return out.T[:n, :out_size]
